# baseline (speedup 1.0000x reference)
import jax
import jax.numpy as jnp
from jax.experimental import pallas as pl
from jax.experimental.pallas import tpu as pltpu

LANES = 128
RB = 8


def _smooth_l1(d):
    a = jnp.abs(d)
    return jnp.where(a < 1.0, 0.5 * d * d, a - 0.5)


def _body(n_batch, n_gt, rows_ref, anc_ref, cls_ref, bbox_ref, ldm_ref, out_ref, gtv_ref):
    j = pl.program_id(0)

    @pl.when(j == 0)
    def _():
        out_ref[...] = jnp.zeros_like(out_ref)

        def fill(r, carry):
            for c in range(12):
                row = rows_ref[pl.ds(r * 12 + c, 1), :]
                gtv_ref[r, c] = jnp.broadcast_to(row, (RB, LANES))
            return carry

        jax.lax.fori_loop(0, n_batch * n_gt, fill, 0)

    x0 = anc_ref[0]
    y0 = anc_ref[1]
    x1 = anc_ref[2]
    y1 = anc_ref[3]
    aw = x1 - x0
    ah = y1 - y0
    a1 = aw * ah
    acx = (x0 + x1) * 0.5
    acy = (y0 + y1) * 0.5
    iaw = 1.0 / aw
    iah = 1.0 / ah
    law = jnp.log(aw)
    lah = jnp.log(ah)

    def batch_body(i, carry):
        base = i * n_gt

        def pair(g):
            ixmin = jnp.maximum(x0, gtv_ref[base + g, 0])
            iymin = jnp.maximum(y0, gtv_ref[base + g, 1])
            ixmax = jnp.minimum(x1, gtv_ref[base + g, 2])
            iymax = jnp.minimum(y1, gtv_ref[base + g, 3])
            dx = jnp.maximum(ixmax - ixmin, 0.0)
            dy = jnp.maximum(iymax - iymin, 0.0)
            inter = dx * dy
            union = (a1 + gtv_ref[base + g, 4]) - inter
            return inter, union

        def trow(g, c):
            return gtv_ref[base + g, 5 + c]

        bi, bu = pair(0)
        tvp = [trow(0, c) for c in range(7)]

        for g in range(1, n_gt):
            inter, union = pair(g)
            w = inter * bu > bi * union
            bi = jnp.where(w, inter, bi)
            bu = jnp.where(w, union, bu)
            tvp = [jnp.where(w, trow(g, c), t) for c, t in enumerate(tvp)]

        tv = []
        for t in tvp:
            u = jax.lax.bitcast_convert_type(t, jnp.uint32)
            tv.append(jax.lax.bitcast_convert_type(u & jnp.uint32(0xFFFF0000), jnp.float32))
            tv.append(jax.lax.bitcast_convert_type(u << 16, jnp.float32))

        m = bi * 2.0 >= bu
        mf = m.astype(jnp.float32)

        c0 = cls_ref[i, 0]
        c1 = cls_ref[i, 1]
        mx = jnp.maximum(c0, c1)
        lse = mx + jnp.log(jnp.exp(c0 - mx) + jnp.exp(c1 - mx))
        ce = lse - jnp.where(m, c1, c0)

        rt0 = (tv[0] - acx) * iaw
        rt1 = (tv[1] - acy) * iah
        rt2 = tv[2] - law
        rt3 = tv[3] - lah
        regs = (_smooth_l1(bbox_ref[i, 0] - rt0) + _smooth_l1(bbox_ref[i, 1] - rt1)
                + _smooth_l1(bbox_ref[i, 2] - rt2) + _smooth_l1(bbox_ref[i, 3] - rt3))
        regs = regs * mf

        ldms = _smooth_l1(ldm_ref[i, 0] - tv[4])
        for c in range(1, 10):
            ldms = ldms + _smooth_l1(ldm_ref[i, c] - tv[4 + c])
        ldms = ldms * mf

        out_ref[i, 0] = out_ref[i, 0] + ce
        out_ref[i, 1] = out_ref[i, 1] + regs
        out_ref[i, 2] = out_ref[i, 2] + ldms
        out_ref[i, 3] = out_ref[i, 3] + mf
        return carry

    jax.lax.fori_loop(0, n_batch, batch_body, 0, unroll=2)


def kernel(pred_cls, pred_bbox, pred_landmarks, anchors, gt_boxes, gt_landmarks):
    n, a, g = pred_cls.shape[0], pred_cls.shape[1], gt_boxes.shape[1]
    ra = a // LANES
    grid = ra // RB

    def prep(lo, hi):
        rah = (hi - lo) // LANES
        ancT = anchors[lo:hi].T.reshape(4, rah, LANES)
        clsT = pred_cls[:, lo:hi].transpose(0, 2, 1).reshape(n, 2, rah, LANES)
        bboxT = pred_bbox[:, lo:hi].transpose(0, 2, 1).reshape(n, 4, rah, LANES)
        ldmT = pred_landmarks[:, lo:hi].transpose(0, 2, 1).reshape(n, 10, rah, LANES)
        return ancT, clsT, bboxT, ldmT

    gw = gt_boxes[..., 2] - gt_boxes[..., 0]
    gh = gt_boxes[..., 3] - gt_boxes[..., 1]
    gtab = jnp.concatenate(
        [gt_boxes,
         (gw * gh)[..., None],
         ((gt_boxes[..., 0] + gt_boxes[..., 2]) * 0.5)[..., None],
         ((gt_boxes[..., 1] + gt_boxes[..., 3]) * 0.5)[..., None],
         jnp.log(gw)[..., None],
         jnp.log(gh)[..., None],
         gt_landmarks], axis=-1).reshape(n * g, 19)
    tgt = gtab[:, 5:19]
    hi = jax.lax.bitcast_convert_type(tgt[:, 0::2].astype(jnp.bfloat16), jnp.uint16).astype(jnp.uint32)
    lo = jax.lax.bitcast_convert_type(tgt[:, 1::2].astype(jnp.bfloat16), jnp.uint16).astype(jnp.uint32)
    packed = jax.lax.bitcast_convert_type((hi << 16) | lo, jnp.float32)
    rows = jnp.concatenate([gtab[:, :5], packed], axis=1)
    rowsb = jnp.broadcast_to(rows.reshape(n * g * 12, 1), (n * g * 12, LANES))

    import functools
    call = pl.pallas_call(
        functools.partial(_body, n, g),
        grid=(grid // 2,),
        in_specs=[
            pl.BlockSpec((n * g * 12, LANES), lambda j: (0, 0)),
            pl.BlockSpec((4, RB, LANES), lambda j: (0, j, 0)),
            pl.BlockSpec((n, 2, RB, LANES), lambda j: (0, 0, j, 0)),
            pl.BlockSpec((n, 4, RB, LANES), lambda j: (0, 0, j, 0)),
            pl.BlockSpec((n, 10, RB, LANES), lambda j: (0, 0, j, 0)),
        ],
        out_specs=pl.BlockSpec((n, 4, RB, LANES), lambda j: (0, 0, 0, 0)),
        out_shape=jax.ShapeDtypeStruct((n, 4, RB, LANES), jnp.float32),
        scratch_shapes=[pltpu.VMEM((n * g, 12, RB, LANES), jnp.float32)],
    )

    out0 = call(rowsb, *prep(0, a // 2))
    out1 = call(rowsb, *prep(a // 2, a))
    out = out0 + out1

    sums = out.sum(axis=(2, 3))
    npos = sums[:, 3]
    cls_loss = jnp.mean(sums[:, 0] / a)
    reg_loss = jnp.mean(sums[:, 1] / (npos * 4.0))
    ldm_loss = jnp.mean(sums[:, 2] / (npos * 10.0))
    total = cls_loss + reg_loss + ldm_loss
    return (total, cls_loss, reg_loss, ldm_loss)


# speedup vs baseline: 1.1171x; 1.1171x over previous
import jax
import jax.numpy as jnp
from jax.experimental import pallas as pl
from jax.experimental.pallas import tpu as pltpu

LANES = 128
RB = 8


def _smooth_l1(d):
    a = jnp.abs(d)
    return jnp.where(a < 1.0, 0.5 * d * d, a - 0.5)


def _body(n_batch, n_gt, rows_ref, anc_ref, cls_ref, bbox_ref, ldm_ref, out_ref, gtv_ref):
    j = pl.program_id(0)

    @pl.when(j == 0)
    def _():
        out_ref[...] = jnp.zeros_like(out_ref)

        def fill(r, carry):
            for c in range(12):
                row = rows_ref[pl.ds(r * 12 + c, 1), :]
                gtv_ref[r, c] = jnp.broadcast_to(row, (RB, LANES))
            return carry

        jax.lax.fori_loop(0, n_batch * n_gt, fill, 0)

    x0 = anc_ref[0]
    y0 = anc_ref[1]
    x1 = anc_ref[2]
    y1 = anc_ref[3]
    aw = x1 - x0
    ah = y1 - y0
    a1 = aw * ah
    acx = (x0 + x1) * 0.5
    acy = (y0 + y1) * 0.5
    iaw = 1.0 / aw
    iah = 1.0 / ah
    law = jnp.log(aw)
    lah = jnp.log(ah)

    def batch_body(i, carry):
        base = i * n_gt

        def pair(g):
            ixmin = jnp.maximum(x0, gtv_ref[base + g, 0])
            iymin = jnp.maximum(y0, gtv_ref[base + g, 1])
            ixmax = jnp.minimum(x1, gtv_ref[base + g, 2])
            iymax = jnp.minimum(y1, gtv_ref[base + g, 3])
            dx = jnp.maximum(ixmax - ixmin, 0.0)
            dy = jnp.maximum(iymax - iymin, 0.0)
            inter = dx * dy
            union = (a1 + gtv_ref[base + g, 4]) - inter
            return inter, union

        def trow(g, c):
            return gtv_ref[base + g, 5 + c]

        bi, bu = pair(0)
        tvp = [trow(0, c) for c in range(7)]

        for g in range(1, n_gt):
            inter, union = pair(g)
            w = inter * bu > bi * union
            bi = jnp.where(w, inter, bi)
            bu = jnp.where(w, union, bu)
            tvp = [jnp.where(w, trow(g, c), t) for c, t in enumerate(tvp)]

        tv = []
        for t in tvp:
            u = jax.lax.bitcast_convert_type(t, jnp.uint32)
            tv.append(jax.lax.bitcast_convert_type(u & jnp.uint32(0xFFFF0000), jnp.float32))
            tv.append(jax.lax.bitcast_convert_type(u << 16, jnp.float32))

        m = bi * 2.0 >= bu
        mf = m.astype(jnp.float32)

        c0 = cls_ref[i, 0]
        c1 = cls_ref[i, 1]
        mx = jnp.maximum(c0, c1)
        lse = mx + jnp.log(jnp.exp(c0 - mx) + jnp.exp(c1 - mx))
        ce = lse - jnp.where(m, c1, c0)

        rt0 = (tv[0] - acx) * iaw
        rt1 = (tv[1] - acy) * iah
        rt2 = tv[2] - law
        rt3 = tv[3] - lah
        regs = (_smooth_l1(bbox_ref[i, 0] - rt0) + _smooth_l1(bbox_ref[i, 1] - rt1)
                + _smooth_l1(bbox_ref[i, 2] - rt2) + _smooth_l1(bbox_ref[i, 3] - rt3))
        regs = regs * mf

        ldms = _smooth_l1(ldm_ref[i, 0] - tv[4])
        for c in range(1, 10):
            ldms = ldms + _smooth_l1(ldm_ref[i, c] - tv[4 + c])
        ldms = ldms * mf

        out_ref[i, 0] = out_ref[i, 0] + ce
        out_ref[i, 1] = out_ref[i, 1] + regs
        out_ref[i, 2] = out_ref[i, 2] + ldms
        out_ref[i, 3] = out_ref[i, 3] + mf
        return carry

    jax.lax.fori_loop(0, n_batch, batch_body, 0, unroll=2)


def kernel(pred_cls, pred_bbox, pred_landmarks, anchors, gt_boxes, gt_landmarks):
    n, a, g = pred_cls.shape[0], pred_cls.shape[1], gt_boxes.shape[1]
    ra = a // LANES
    grid = ra // RB

    def prep(lo, hi):
        rah = (hi - lo) // LANES
        ancT = anchors[lo:hi].T.reshape(4, rah, LANES)
        clsT = pred_cls[:, lo:hi].transpose(0, 2, 1).reshape(n, 2, rah, LANES)
        bboxT = pred_bbox[:, lo:hi].transpose(0, 2, 1).reshape(n, 4, rah, LANES)
        ldmT = pred_landmarks[:, lo:hi].transpose(0, 2, 1).reshape(n, 10, rah, LANES)
        return ancT, clsT, bboxT, ldmT

    gw = gt_boxes[..., 2] - gt_boxes[..., 0]
    gh = gt_boxes[..., 3] - gt_boxes[..., 1]
    gtab = jnp.concatenate(
        [gt_boxes,
         (gw * gh)[..., None],
         ((gt_boxes[..., 0] + gt_boxes[..., 2]) * 0.5)[..., None],
         ((gt_boxes[..., 1] + gt_boxes[..., 3]) * 0.5)[..., None],
         jnp.log(gw)[..., None],
         jnp.log(gh)[..., None],
         gt_landmarks], axis=-1).reshape(n * g, 19)
    tgt = gtab[:, 5:19]
    hi = jax.lax.bitcast_convert_type(tgt[:, 0::2].astype(jnp.bfloat16), jnp.uint16).astype(jnp.uint32)
    lo = jax.lax.bitcast_convert_type(tgt[:, 1::2].astype(jnp.bfloat16), jnp.uint16).astype(jnp.uint32)
    packed = jax.lax.bitcast_convert_type((hi << 16) | lo, jnp.float32)
    rows = jnp.concatenate([gtab[:, :5], packed], axis=1)
    rowsb = jnp.broadcast_to(rows.reshape(n * g * 12, 1), (n * g * 12, LANES))

    import functools
    call = pl.pallas_call(
        functools.partial(_body, n, g),
        grid=(grid,),
        in_specs=[
            pl.BlockSpec((n * g * 12, LANES), lambda j: (0, 0)),
            pl.BlockSpec((4, RB, LANES), lambda j: (0, j, 0)),
            pl.BlockSpec((n, 2, RB, LANES), lambda j: (0, 0, j, 0)),
            pl.BlockSpec((n, 4, RB, LANES), lambda j: (0, 0, j, 0)),
            pl.BlockSpec((n, 10, RB, LANES), lambda j: (0, 0, j, 0)),
        ],
        out_specs=pl.BlockSpec((n, 4, RB, LANES), lambda j: (0, 0, 0, 0)),
        out_shape=jax.ShapeDtypeStruct((n, 4, RB, LANES), jnp.float32),
        scratch_shapes=[pltpu.VMEM((n * g, 12, RB, LANES), jnp.float32)],
    )

    out = call(rowsb, *prep(0, a))

    sums = out.sum(axis=(2, 3))
    npos = sums[:, 3]
    cls_loss = jnp.mean(sums[:, 0] / a)
    reg_loss = jnp.mean(sums[:, 1] / (npos * 4.0))
    ldm_loss = jnp.mean(sums[:, 2] / (npos * 10.0))
    total = cls_loss + reg_loss + ldm_loss
    return (total, cls_loss, reg_loss, ldm_loss)


# speedup vs baseline: 1.1231x; 1.0054x over previous
import jax
import jax.numpy as jnp
from jax.experimental import pallas as pl
from jax.experimental.pallas import tpu as pltpu

LANES = 128
RB = 8


def _smooth_l1(d):
    a = jnp.abs(d)
    return jnp.where(a < 1.0, 0.5 * d * d, a - 0.5)


def _body(n_batch, n_gt, rows_ref, anc_ref, cls_ref, bbox_ref, ldm_ref, out_ref, gtv_ref):
    j = pl.program_id(0)

    @pl.when(j == 0)
    def _():
        out_ref[...] = jnp.zeros_like(out_ref)

        def fill(r, carry):
            for c in range(12):
                row = rows_ref[pl.ds(r * 12 + c, 1), :]
                gtv_ref[r, c] = jnp.broadcast_to(row, (RB, LANES))
            return carry

        jax.lax.fori_loop(0, n_batch * n_gt, fill, 0)

    x0 = anc_ref[0]
    y0 = anc_ref[1]
    x1 = anc_ref[2]
    y1 = anc_ref[3]
    aw = x1 - x0
    ah = y1 - y0
    a1 = aw * ah
    acx = (x0 + x1) * 0.5
    acy = (y0 + y1) * 0.5
    iaw = 1.0 / aw
    iah = 1.0 / ah
    law = jnp.log(aw)
    lah = jnp.log(ah)

    def batch_body(i, carry):
        base = i * n_gt

        def pair(g):
            ixmin = jnp.maximum(x0, gtv_ref[base + g, 0])
            iymin = jnp.maximum(y0, gtv_ref[base + g, 1])
            ixmax = jnp.minimum(x1, gtv_ref[base + g, 2])
            iymax = jnp.minimum(y1, gtv_ref[base + g, 3])
            dx = jnp.maximum(ixmax - ixmin, 0.0)
            dy = jnp.maximum(iymax - iymin, 0.0)
            inter = dx * dy
            union = (a1 + gtv_ref[base + g, 4]) - inter
            return inter, union

        def trow(g, c):
            return gtv_ref[base + g, 5 + c]

        bi, bu = pair(0)
        tvp = [trow(0, c) for c in range(7)]

        for g in range(1, n_gt):
            inter, union = pair(g)
            w = inter * bu > bi * union
            bi = jnp.where(w, inter, bi)
            bu = jnp.where(w, union, bu)
            tvp = [jnp.where(w, trow(g, c), t) for c, t in enumerate(tvp)]

        tv = []
        for t in tvp:
            u = jax.lax.bitcast_convert_type(t, jnp.uint32)
            tv.append(jax.lax.bitcast_convert_type(u & jnp.uint32(0xFFFF0000), jnp.float32))
            tv.append(jax.lax.bitcast_convert_type(u << 16, jnp.float32))

        m = bi * 2.0 >= bu
        mf = m.astype(jnp.float32)

        c0 = cls_ref[i, 0]
        c1 = cls_ref[i, 1]
        mx = jnp.maximum(c0, c1)
        lse = mx + jnp.log(jnp.exp(c0 - mx) + jnp.exp(c1 - mx))
        ce = lse - jnp.where(m, c1, c0)

        rt0 = (tv[0] - acx) * iaw
        rt1 = (tv[1] - acy) * iah
        rt2 = tv[2] - law
        rt3 = tv[3] - lah
        regs = (_smooth_l1(bbox_ref[i, 0] - rt0) + _smooth_l1(bbox_ref[i, 1] - rt1)
                + _smooth_l1(bbox_ref[i, 2] - rt2) + _smooth_l1(bbox_ref[i, 3] - rt3))
        regs = regs * mf

        ldms = _smooth_l1(ldm_ref[i, 0] - tv[4])
        for c in range(1, 10):
            ldms = ldms + _smooth_l1(ldm_ref[i, c] - tv[4 + c])
        ldms = ldms * mf

        out_ref[i, 0] = out_ref[i, 0] + ce
        out_ref[i, 1] = out_ref[i, 1] + regs
        out_ref[i, 2] = out_ref[i, 2] + ldms
        out_ref[i, 3] = out_ref[i, 3] + mf
        return carry

    jax.lax.fori_loop(0, n_batch, batch_body, 0, unroll=4)


def kernel(pred_cls, pred_bbox, pred_landmarks, anchors, gt_boxes, gt_landmarks):
    n, a, g = pred_cls.shape[0], pred_cls.shape[1], gt_boxes.shape[1]
    ra = a // LANES
    grid = ra // RB

    def prep(lo, hi):
        rah = (hi - lo) // LANES
        ancT = anchors[lo:hi].T.reshape(4, rah, LANES)
        clsT = pred_cls[:, lo:hi].transpose(0, 2, 1).reshape(n, 2, rah, LANES)
        bboxT = pred_bbox[:, lo:hi].transpose(0, 2, 1).reshape(n, 4, rah, LANES)
        ldmT = pred_landmarks[:, lo:hi].transpose(0, 2, 1).reshape(n, 10, rah, LANES)
        return ancT, clsT, bboxT, ldmT

    gw = gt_boxes[..., 2] - gt_boxes[..., 0]
    gh = gt_boxes[..., 3] - gt_boxes[..., 1]
    gtab = jnp.concatenate(
        [gt_boxes,
         (gw * gh)[..., None],
         ((gt_boxes[..., 0] + gt_boxes[..., 2]) * 0.5)[..., None],
         ((gt_boxes[..., 1] + gt_boxes[..., 3]) * 0.5)[..., None],
         jnp.log(gw)[..., None],
         jnp.log(gh)[..., None],
         gt_landmarks], axis=-1).reshape(n * g, 19)
    tgt = gtab[:, 5:19]
    hi = jax.lax.bitcast_convert_type(tgt[:, 0::2].astype(jnp.bfloat16), jnp.uint16).astype(jnp.uint32)
    lo = jax.lax.bitcast_convert_type(tgt[:, 1::2].astype(jnp.bfloat16), jnp.uint16).astype(jnp.uint32)
    packed = jax.lax.bitcast_convert_type((hi << 16) | lo, jnp.float32)
    rows = jnp.concatenate([gtab[:, :5], packed], axis=1)
    rowsb = jnp.broadcast_to(rows.reshape(n * g * 12, 1), (n * g * 12, LANES))

    import functools
    call = pl.pallas_call(
        functools.partial(_body, n, g),
        grid=(grid,),
        in_specs=[
            pl.BlockSpec((n * g * 12, LANES), lambda j: (0, 0)),
            pl.BlockSpec((4, RB, LANES), lambda j: (0, j, 0)),
            pl.BlockSpec((n, 2, RB, LANES), lambda j: (0, 0, j, 0)),
            pl.BlockSpec((n, 4, RB, LANES), lambda j: (0, 0, j, 0)),
            pl.BlockSpec((n, 10, RB, LANES), lambda j: (0, 0, j, 0)),
        ],
        out_specs=pl.BlockSpec((n, 4, RB, LANES), lambda j: (0, 0, 0, 0)),
        out_shape=jax.ShapeDtypeStruct((n, 4, RB, LANES), jnp.float32),
        scratch_shapes=[pltpu.VMEM((n * g, 12, RB, LANES), jnp.float32)],
    )

    out = call(rowsb, *prep(0, a))

    sums = out.sum(axis=(2, 3))
    npos = sums[:, 3]
    cls_loss = jnp.mean(sums[:, 0] / a)
    reg_loss = jnp.mean(sums[:, 1] / (npos * 4.0))
    ldm_loss = jnp.mean(sums[:, 2] / (npos * 10.0))
    total = cls_loss + reg_loss + ldm_loss
    return (total, cls_loss, reg_loss, ldm_loss)
